# baseline (speedup 1.0000x reference)
import functools

import jax
import jax.numpy as jnp
from jax import lax
from jax.experimental import pallas as pl
from jax.experimental.pallas import tpu as pltpu

N = 10000
E = 320000
DIN = 128
DE = 16
DOUT = 128

ROW_BLK = 1000


def _prep_body(nf_ref, w1_ref, a12_ref, m1_ref, s12_ref):
    nf = nf_ref[...]
    m1_ref[...] = lax.dot_general(nf, w1_ref[...], (((1,), (1,)), ((), ())),
                                  preferred_element_type=jnp.float32)
    s12_ref[...] = jnp.dot(nf, a12_ref[...], preferred_element_type=jnp.float32)


def _tc_prep(nf, w1, a12):
    grid = (N // ROW_BLK,)
    return pl.pallas_call(
        _prep_body,
        grid=grid,
        in_specs=[
            pl.BlockSpec((ROW_BLK, DIN), lambda i: (i, 0)),
            pl.BlockSpec((DOUT, DIN), lambda i: (0, 0)),
            pl.BlockSpec((DIN, 2), lambda i: (0, 0)),
        ],
        out_specs=[
            pl.BlockSpec((ROW_BLK, DOUT), lambda i: (i, 0)),
            pl.BlockSpec((ROW_BLK, 2), lambda i: (i, 0)),
        ],
        out_shape=[
            jax.ShapeDtypeStruct((N, DOUT), jnp.float32),
            jax.ShapeDtypeStruct((N, 2), jnp.float32),
        ],
    )(nf, w1, a12)


def _apply_body(nf_ref, z1_ref, q_ref, w2_ref, wa1_ref, wa2_ref, wb_ref, out_ref):
    z = z1_ref[...] + lax.dot_general(
        q_ref[...], w2_ref[...], (((1,), (1,)), ((), ())),
        preferred_element_type=jnp.float32)
    acc = lax.dot_general(nf_ref[...], wa1_ref[...], (((1,), (1,)), ((), ())),
                          preferred_element_type=jnp.float32)
    acc = acc + lax.dot_general(z, wa2_ref[...], (((1,), (1,)), ((), ())),
                                preferred_element_type=jnp.float32)
    out_ref[...] = jnp.maximum(acc + wb_ref[...], 0.0)


def _tc_apply(nf, z1, q, w2, wa1, wa2, wb):
    grid = (N // ROW_BLK,)
    return pl.pallas_call(
        _apply_body,
        grid=grid,
        in_specs=[
            pl.BlockSpec((ROW_BLK, DIN), lambda i: (i, 0)),
            pl.BlockSpec((ROW_BLK, DOUT), lambda i: (i, 0)),
            pl.BlockSpec((ROW_BLK, DE), lambda i: (i, 0)),
            pl.BlockSpec((DOUT, DE), lambda i: (0, 0)),
            pl.BlockSpec((DOUT, DIN), lambda i: (0, 0)),
            pl.BlockSpec((DOUT, DOUT), lambda i: (0, 0)),
            pl.BlockSpec((1, DOUT), lambda i: (0, 0)),
        ],
        out_specs=pl.BlockSpec((ROW_BLK, DOUT), lambda i: (i, 0)),
        out_shape=jax.ShapeDtypeStruct((N, DOUT), jnp.float32),
    )(nf, z1, q, w2, wa1, wa2, wb)


@jax.jit
def kernel(nfeats, efeats, edge_index, lin_w, attn_w, attn_b, Wapply_w, Wapply_b):
    nf = nfeats.reshape(N, DIN)
    ef = efeats.reshape(E, DE)
    src = edge_index[0]
    dst = edge_index[1]

    w1 = lin_w[:, :DIN]
    w2 = lin_w[:, DIN:]
    a12 = attn_w.reshape(2, DIN).T
    wa1 = Wapply_w[:, :DIN]
    wa2 = Wapply_w[:, DIN:]

    m1, s12 = _tc_prep(nf, w1, a12)
    s1 = s12[:, 0]
    s2 = s12[:, 1]

    e = s1[src] + s2[dst] + attn_b[0]
    e = jnp.where(e >= 0, e, 0.01 * e)
    ex = jnp.exp(e)
    denom = jax.ops.segment_sum(ex, dst, num_segments=N)
    alpha = ex / denom[dst]
    z1 = jax.ops.segment_sum(alpha[:, None] * m1[src], dst, num_segments=N)
    q = jax.ops.segment_sum(alpha[:, None] * ef, dst, num_segments=N)

    out = _tc_apply(nf, z1, q, w2, wa1, wa2, Wapply_b.reshape(1, DOUT))
    return out.reshape(N, 1, DOUT)


# speedup vs baseline: 6.3058x; 6.3058x over previous
import functools

import jax
import jax.numpy as jnp
from jax import lax
from jax.experimental import pallas as pl
from jax.experimental.pallas import tpu as pltpu
from jax.experimental.pallas import tpu_sc as plsc

N = 10000
E = 320000
DIN = 128
DE = 16
DOUT = 128

ROW_BLK = 1000

NC = 2
NS = 16
NPAD = 10240
NSL = NPAD // NS
CH = 80
DH = DOUT // 2
EPT = E // NS
RPT = EPT // CH
SB = 25
NSUP = RPT // SB


def _z16():
    return jnp.zeros((16,), jnp.float32)


def _prep_body(nf_ref, w1_ref, a12_ref, ab_ref, m1_ref, s12_ref):
    nf = nf_ref[...]
    m1_ref[...] = lax.dot_general(nf, w1_ref[...], (((1,), (1,)), ((), ())),
                                  preferred_element_type=jnp.float32)
    s12_ref[...] = jnp.dot(nf, a12_ref[...],
                           preferred_element_type=jnp.float32) + ab_ref[...]


def _tc_prep(nf, w1, a12, ab):
    return pl.pallas_call(
        _prep_body,
        grid=(N // ROW_BLK,),
        in_specs=[
            pl.BlockSpec((ROW_BLK, DIN), lambda i: (i, 0)),
            pl.BlockSpec((DOUT, DIN), lambda i: (0, 0)),
            pl.BlockSpec((DIN, 2), lambda i: (0, 0)),
            pl.BlockSpec((1, 2), lambda i: (0, 0)),
        ],
        out_specs=[
            pl.BlockSpec((ROW_BLK, DOUT), lambda i: (i, 0)),
            pl.BlockSpec((ROW_BLK, 2), lambda i: (i, 0)),
        ],
        out_shape=[
            jax.ShapeDtypeStruct((N, DOUT), jnp.float32),
            jax.ShapeDtypeStruct((N, 2), jnp.float32),
        ],
    )(nf, w1, a12, ab)


def _apply_body(nf_ref, z1p_ref, qp_ref, den_ref, w2_ref, wa1_ref, wa2_ref,
                wb_ref, out_ref):
    zt = jnp.concatenate([z1p_ref[0], z1p_ref[1]], axis=-1)
    zt = zt + lax.dot_general(qp_ref[0], w2_ref[...], (((1,), (1,)), ((), ())),
                              preferred_element_type=jnp.float32)
    den = den_ref[...]
    z = zt / jnp.where(den > 0.0, den, 1.0)
    acc = lax.dot_general(nf_ref[...], wa1_ref[...], (((1,), (1,)), ((), ())),
                          preferred_element_type=jnp.float32)
    acc = acc + lax.dot_general(z, wa2_ref[...], (((1,), (1,)), ((), ())),
                                preferred_element_type=jnp.float32)
    out_ref[...] = jnp.maximum(acc + wb_ref[...], 0.0)


def _tc_apply(nf, z1p, qp, den_col, w2, wa1, wa2, wb):
    return pl.pallas_call(
        _apply_body,
        grid=(N // ROW_BLK,),
        in_specs=[
            pl.BlockSpec((ROW_BLK, DIN), lambda i: (i, 0)),
            pl.BlockSpec((NC, ROW_BLK, DH), lambda i: (0, i, 0)),
            pl.BlockSpec((NC, ROW_BLK, DE), lambda i: (0, i, 0)),
            pl.BlockSpec((ROW_BLK, 1), lambda i: (i, 0)),
            pl.BlockSpec((DOUT, DE), lambda i: (0, 0)),
            pl.BlockSpec((DOUT, DIN), lambda i: (0, 0)),
            pl.BlockSpec((DOUT, DOUT), lambda i: (0, 0)),
            pl.BlockSpec((1, DOUT), lambda i: (0, 0)),
        ],
        out_specs=pl.BlockSpec((ROW_BLK, DOUT), lambda i: (i, 0)),
        out_shape=jax.ShapeDtypeStruct((N, DOUT), jnp.float32),
    )(nf, z1p, qp, den_col, w2, wa1, wa2, wb)


def _sc_pass(srcB, dstB, s1p, s2p, m1h, ef):
    mesh = plsc.VectorSubcoreMesh(core_axis_name="c", subcore_axis_name="s")

    @functools.partial(
        pl.kernel,
        out_type=[
            jax.ShapeDtypeStruct((NC, NPAD, DH), jnp.float32),
            jax.ShapeDtypeStruct((NC, NPAD, DE), jnp.float32),
            jax.ShapeDtypeStruct((NC, 1, NPAD), jnp.float32),
        ],
        mesh=mesh,
        compiler_params=pltpu.CompilerParams(
            needs_layout_passes=False, use_tc_tiling_on_sc=False),
        scratch_types=[
            pltpu.VMEM((SB, CH), jnp.int32),
            pltpu.VMEM((SB, CH), jnp.int32),
            pltpu.VMEM((NPAD,), jnp.float32),
            pltpu.VMEM((NPAD,), jnp.float32),
            pltpu.VMEM((CH,), jnp.float32),
            pltpu.VMEM((CH, DH), jnp.float32),
            pltpu.VMEM((CH, DE), jnp.float32),
            pltpu.VMEM((NSL // 8, DH), jnp.float32),
            pltpu.VMEM((NSL // 2, DE), jnp.float32),
            pltpu.VMEM((NSL,), jnp.float32),
            pltpu.VMEM_SHARED((NPAD, DH), jnp.float32),
            pltpu.VMEM_SHARED((NPAD, DE), jnp.float32),
            pltpu.VMEM_SHARED((NPAD,), jnp.float32),
            pltpu.SemaphoreType.DMA,
        ],
    )
    def k(src_hbm, dst_hbm, s1_hbm, s2_hbm, m1h_hbm, ef_hbm,
          z1p_hbm, qp_hbm, dp_hbm,
          src_v, dst_v, s1_v, s2_v, ex_v, rows_v, ef_v,
          cp_v, qz_v, dz_v, z1_sh, q_sh, den_sh, sem):
        c = lax.axis_index("c")
        s = lax.axis_index("s")
        pltpu.sync_copy(s1_hbm, s1_v)
        pltpu.sync_copy(s2_hbm, s2_v)

        def zrow_body(r, carry):
            for j in range(DH // 16):
                cp_v[r, pl.ds(j * 16, 16)] = _z16()
            return carry
        lax.fori_loop(0, NSL // 8, zrow_body, 0)

        def zq_body(r, carry):
            qz_v[r, :] = _z16()
            return carry
        lax.fori_loop(0, NSL // 2, zq_body, 0)

        def zd_body(i, carry):
            dz_v[pl.ds(i * 16, 16)] = _z16()
            return carry
        lax.fori_loop(0, NSL // 16, zd_body, 0)

        for t in range(8):
            pltpu.sync_copy(
                cp_v, z1_sh.at[pl.ds(s * NSL + t * (NSL // 8), NSL // 8)])
        for t in range(2):
            pltpu.sync_copy(
                qz_v, q_sh.at[pl.ds(s * NSL + t * (NSL // 2), NSL // 2)])
        pltpu.sync_copy(dz_v, den_sh.at[pl.ds(s * NSL, NSL)])
        plsc.subcore_barrier()

        def sup_body(m, carry0):
            pltpu.sync_copy(src_hbm.at[s, pl.ds(m * SB, SB)], src_v)
            pltpu.sync_copy(dst_hbm.at[s, pl.ds(m * SB, SB)], dst_v)

            def chunk_body(kk, carry):
                for j in range(CH // 16):
                    i_s = src_v[kk, pl.ds(j * 16, 16)]
                    i_d = dst_v[kk, pl.ds(j * 16, 16)]
                    v = (plsc.load_gather(s1_v, [i_s])
                         + plsc.load_gather(s2_v, [i_d]))
                    v = jnp.where(v >= 0.0, v, v * 0.01)
                    ex_v[pl.ds(j * 16, 16)] = jnp.exp(v)
                pltpu.async_copy(
                    m1h_hbm.at[c].at[src_v.at[kk]], rows_v, sem).wait()
                pltpu.sync_copy(
                    ef_hbm.at[pl.ds(s * EPT + (m * SB + kk) * CH, CH)], ef_v)

                def scale_body(g, carry2):
                    av = ex_v[pl.ds(g * 16, 16)]
                    for l in range(16):
                        a = av[l]
                        i = g * 16 + l
                        for j in range(DH // 16):
                            sl = pl.ds(j * 16, 16)
                            rows_v[i, sl] = rows_v[i, sl] * a
                        ef_v[i, :] = ef_v[i, :] * a
                    return carry2
                lax.fori_loop(0, CH // 16, scale_body, 0)

                pltpu.sync_copy(rows_v, z1_sh.at[dst_v.at[kk]], add=True)
                pltpu.sync_copy(ef_v, q_sh.at[dst_v.at[kk]], add=True)
                pltpu.sync_copy(ex_v, den_sh.at[dst_v.at[kk]], add=True)
                return carry
            lax.fori_loop(0, SB, chunk_body, 0)
            return carry0
        lax.fori_loop(0, NSUP, sup_body, 0)
        plsc.subcore_barrier()

        for t in range(8):
            sl = pl.ds(s * NSL + t * (NSL // 8), NSL // 8)
            pltpu.sync_copy(z1_sh.at[sl], cp_v)
            pltpu.sync_copy(cp_v, z1p_hbm.at[c, sl])
        for t in range(2):
            sl = pl.ds(s * NSL + t * (NSL // 2), NSL // 2)
            pltpu.sync_copy(q_sh.at[sl], qz_v)
            pltpu.sync_copy(qz_v, qp_hbm.at[c, sl])
        pltpu.sync_copy(den_sh.at[pl.ds(s * NSL, NSL)], dz_v)
        pltpu.sync_copy(dz_v, dp_hbm.at[c, 0, pl.ds(s * NSL, NSL)])

    return k(srcB, dstB, s1p, s2p, m1h, ef)


@jax.jit
def kernel(nfeats, efeats, edge_index, lin_w, attn_w, attn_b, Wapply_w, Wapply_b):
    nf = nfeats.reshape(N, DIN)
    ef = efeats.reshape(E, DE)
    srcB = edge_index[0].reshape(NS, RPT, CH)
    dstB = edge_index[1].reshape(NS, RPT, CH)

    w1 = lin_w[:, :DIN]
    w2 = lin_w[:, DIN:]
    a12 = attn_w.reshape(2, DIN).T
    ab = jnp.stack([attn_b[0], jnp.float32(0.0)]).reshape(1, 2)
    wa1 = Wapply_w[:, :DIN]
    wa2 = Wapply_w[:, DIN:]

    m1, s12 = _tc_prep(nf, w1, a12, ab)
    s1p = jnp.pad(s12[:, 0], (0, NPAD - N))
    s2p = jnp.pad(s12[:, 1], (0, NPAD - N))
    m1h = jnp.stack([m1[:, :DH], m1[:, DH:]])

    z1p, qp, dp = _sc_pass(srcB, dstB, s1p, s2p, m1h, ef)
    den_col = dp[0, 0, :N].reshape(N, 1)

    out = _tc_apply(nf, z1p, qp, den_col, w2, wa1, wa2,
                    Wapply_b.reshape(1, DOUT))
    return out.reshape(N, 1, DOUT)


# speedup vs baseline: 17.8533x; 2.8312x over previous
import functools

import jax
import jax.numpy as jnp
from jax import lax
from jax.experimental import pallas as pl
from jax.experimental.pallas import tpu as pltpu
from jax.experimental.pallas import tpu_sc as plsc

N = 10000
E = 320000
DIN = 128
DE = 16
DOUT = 128

ROW_BLK = 1000

NC = 2
NS = 16
NPAD = 10240
NSL = NPAD // NS
CH = 80
DH = DOUT // 2
EPT = E // NS
RPT = EPT // CH
SB = 50
NSUP = RPT // SB


def _z16():
    return jnp.zeros((16,), jnp.float32)


def _prep_body(nf_ref, w1_ref, a12_ref, ab_ref, m1_ref, s12_ref):
    nf = nf_ref[...]
    m1_ref[...] = lax.dot_general(nf, w1_ref[...], (((1,), (1,)), ((), ())),
                                  preferred_element_type=jnp.float32)
    s12_ref[...] = jnp.dot(nf, a12_ref[...],
                           preferred_element_type=jnp.float32) + ab_ref[...]


def _tc_prep(nf, w1, a12, ab):
    return pl.pallas_call(
        _prep_body,
        grid=(N // ROW_BLK,),
        in_specs=[
            pl.BlockSpec((ROW_BLK, DIN), lambda i: (i, 0)),
            pl.BlockSpec((DOUT, DIN), lambda i: (0, 0)),
            pl.BlockSpec((DIN, 2), lambda i: (0, 0)),
            pl.BlockSpec((1, 2), lambda i: (0, 0)),
        ],
        out_specs=[
            pl.BlockSpec((ROW_BLK, DOUT), lambda i: (i, 0)),
            pl.BlockSpec((ROW_BLK, 2), lambda i: (i, 0)),
        ],
        out_shape=[
            jax.ShapeDtypeStruct((N, DOUT), jnp.float32),
            jax.ShapeDtypeStruct((N, 2), jnp.float32),
        ],
    )(nf, w1, a12, ab)


def _apply_body(nf_ref, z1p_ref, qp_ref, den_ref, w2_ref, wa1_ref, wa2_ref,
                wb_ref, out_ref):
    zt = jnp.concatenate([z1p_ref[0], z1p_ref[1]], axis=-1)
    zt = zt + lax.dot_general(qp_ref[0], w2_ref[...], (((1,), (1,)), ((), ())),
                              preferred_element_type=jnp.float32)
    den = den_ref[...]
    z = zt / jnp.where(den > 0.0, den, 1.0)
    acc = lax.dot_general(nf_ref[...], wa1_ref[...], (((1,), (1,)), ((), ())),
                          preferred_element_type=jnp.float32)
    acc = acc + lax.dot_general(z, wa2_ref[...], (((1,), (1,)), ((), ())),
                                preferred_element_type=jnp.float32)
    out_ref[...] = jnp.maximum(acc + wb_ref[...], 0.0)


def _tc_apply(nf, z1p, qp, den_col, w2, wa1, wa2, wb):
    return pl.pallas_call(
        _apply_body,
        grid=(N // ROW_BLK,),
        in_specs=[
            pl.BlockSpec((ROW_BLK, DIN), lambda i: (i, 0)),
            pl.BlockSpec((NC, ROW_BLK, DH), lambda i: (0, i, 0)),
            pl.BlockSpec((NC, ROW_BLK, DE), lambda i: (0, i, 0)),
            pl.BlockSpec((ROW_BLK, 1), lambda i: (i, 0)),
            pl.BlockSpec((DOUT, DE), lambda i: (0, 0)),
            pl.BlockSpec((DOUT, DIN), lambda i: (0, 0)),
            pl.BlockSpec((DOUT, DOUT), lambda i: (0, 0)),
            pl.BlockSpec((1, DOUT), lambda i: (0, 0)),
        ],
        out_specs=pl.BlockSpec((ROW_BLK, DOUT), lambda i: (i, 0)),
        out_shape=jax.ShapeDtypeStruct((N, DOUT), jnp.float32),
    )(nf, z1p, qp, den_col, w2, wa1, wa2, wb)


def _sc_pass(srcB, dstB, s1p, s2p, m1h, ef):
    mesh = plsc.VectorSubcoreMesh(core_axis_name="c", subcore_axis_name="s")

    @functools.partial(
        pl.kernel,
        out_type=[
            jax.ShapeDtypeStruct((NC, NPAD, DH), jnp.float32),
            jax.ShapeDtypeStruct((NC, NPAD, DE), jnp.float32),
            jax.ShapeDtypeStruct((NC, 1, NPAD), jnp.float32),
        ],
        mesh=mesh,
        compiler_params=pltpu.CompilerParams(
            needs_layout_passes=False, use_tc_tiling_on_sc=False),
        scratch_types=[
            pltpu.VMEM((SB, CH), jnp.int32),
            pltpu.VMEM((SB, CH), jnp.int32),
            pltpu.VMEM((NPAD,), jnp.float32),
            pltpu.VMEM((NPAD,), jnp.float32),
            pltpu.VMEM((CH, DH), jnp.float32),
            pltpu.VMEM((CH, DH), jnp.float32),
            pltpu.VMEM((CH, DH), jnp.float32),
            pltpu.VMEM((CH, DH), jnp.float32),
            pltpu.VMEM((CH, DE), jnp.float32),
            pltpu.VMEM((CH, DE), jnp.float32),
            pltpu.VMEM((CH, DE), jnp.float32),
            pltpu.VMEM((CH, DE), jnp.float32),
            pltpu.VMEM((CH,), jnp.float32),
            pltpu.VMEM((CH,), jnp.float32),
            pltpu.VMEM((NSL // 8, DH), jnp.float32),
            pltpu.VMEM((NSL // 2, DE), jnp.float32),
            pltpu.VMEM((NSL,), jnp.float32),
            pltpu.VMEM_SHARED((NPAD, DH), jnp.float32),
            pltpu.VMEM_SHARED((NPAD, DE), jnp.float32),
            pltpu.VMEM_SHARED((NPAD,), jnp.float32),
            pltpu.SemaphoreType.DMA,
            pltpu.SemaphoreType.DMA,
            pltpu.SemaphoreType.DMA,
            pltpu.SemaphoreType.DMA,
        ],
    )
    def k(src_hbm, dst_hbm, s1_hbm, s2_hbm, m1h_hbm, ef_hbm,
          z1p_hbm, qp_hbm, dp_hbm,
          src_v, dst_v, s1_v, s2_v,
          rows0_v, rows1_v, srow0_v, srow1_v,
          ef0_v, ef1_v, sef0_v, sef1_v, ex0_v, ex1_v,
          cp_v, qz_v, dz_v, z1_sh, q_sh, den_sh,
          isem0, isem1, osem0, osem1):
        c = lax.axis_index("c")
        s = lax.axis_index("s")
        pltpu.sync_copy(s1_hbm, s1_v)
        pltpu.sync_copy(s2_hbm, s2_v)

        def zrow_body(r, carry):
            for j in range(DH // 16):
                cp_v[r, pl.ds(j * 16, 16)] = _z16()
            return carry
        lax.fori_loop(0, NSL // 8, zrow_body, 0)

        def zq_body(r, carry):
            qz_v[r, :] = _z16()
            return carry
        lax.fori_loop(0, NSL // 2, zq_body, 0)

        def zd_body(i, carry):
            dz_v[pl.ds(i * 16, 16)] = _z16()
            return carry
        lax.fori_loop(0, NSL // 16, zd_body, 0)

        for t in range(8):
            pltpu.sync_copy(
                cp_v, z1_sh.at[pl.ds(s * NSL + t * (NSL // 8), NSL // 8)])
        for t in range(2):
            pltpu.sync_copy(
                qz_v, q_sh.at[pl.ds(s * NSL + t * (NSL // 2), NSL // 2)])
        pltpu.sync_copy(dz_v, den_sh.at[pl.ds(s * NSL, NSL)])
        plsc.subcore_barrier()

        bufs = ((rows0_v, srow0_v, ef0_v, sef0_v, ex0_v, isem0, osem0),
                (rows1_v, srow1_v, ef1_v, sef1_v, ex1_v, isem1, osem1))

        def sup_body(m, carry0):
            pltpu.sync_copy(src_hbm.at[s, pl.ds(m * SB, SB)], src_v)
            pltpu.sync_copy(dst_hbm.at[s, pl.ds(m * SB, SB)], dst_v)

            def issue_in(kk, b):
                rows, _, efb, _, _, isem, _ = bufs[b]
                pltpu.async_copy(m1h_hbm.at[c].at[src_v.at[kk]], rows, isem)

                @pl.when(c == 0)
                def _():
                    pltpu.async_copy(
                        ef_hbm.at[pl.ds(s * EPT + (m * SB + kk) * CH, CH)],
                        efb, isem)

            def wait_in(kk, b):
                rows, _, efb, _, _, isem, _ = bufs[b]
                pltpu.make_async_copy(
                    m1h_hbm.at[c].at[src_v.at[kk]], rows, isem).wait()

                @pl.when(c == 0)
                def _():
                    pltpu.make_async_copy(
                        ef_hbm.at[pl.ds(s * EPT + (m * SB + kk) * CH, CH)],
                        efb, isem).wait()

            def issue_out(kk, b):
                _, srow, _, sefb, exb, _, osem = bufs[b]
                pltpu.async_copy(srow, z1_sh.at[dst_v.at[kk]], osem, add=True)

                @pl.when(c == 0)
                def _():
                    pltpu.async_copy(sefb, q_sh.at[dst_v.at[kk]], osem,
                                     add=True)

                @pl.when(c == 1)
                def _():
                    pltpu.async_copy(exb, den_sh.at[dst_v.at[kk]], osem,
                                     add=True)

            def drain_out(kk, b):
                _, srow, _, sefb, exb, _, osem = bufs[b]
                pltpu.make_async_copy(
                    srow, z1_sh.at[dst_v.at[kk]], osem).wait()

                @pl.when(c == 0)
                def _():
                    pltpu.make_async_copy(
                        sefb, q_sh.at[dst_v.at[kk]], osem).wait()

                @pl.when(c == 1)
                def _():
                    pltpu.make_async_copy(
                        exb, den_sh.at[dst_v.at[kk]], osem).wait()

            issue_in(0, 0)
            issue_in(1, 1)

            def pair_body(t, carry):
                for b in range(2):
                    kk = t * 2 + b
                    rows, srow, efb, sefb, exb, isem, osem = bufs[b]

                    @pl.when(t >= 1)
                    def _():
                        drain_out(kk - 2, b)

                    wait_in(kk, b)

                    for j in range(CH // 16):
                        i_s = src_v[kk, pl.ds(j * 16, 16)]
                        i_d = dst_v[kk, pl.ds(j * 16, 16)]
                        v = (plsc.load_gather(s1_v, [i_s])
                             + plsc.load_gather(s2_v, [i_d]))
                        v = jnp.where(v >= 0.0, v, v * 0.01)
                        exb[pl.ds(j * 16, 16)] = jnp.exp(v)

                    def scale_body(g, carry2):
                        av = exb[pl.ds(g * 16, 16)]
                        for l in range(16):
                            a = av[l]
                            i = g * 16 + l
                            for j in range(DH // 16):
                                sl = pl.ds(j * 16, 16)
                                srow[i, sl] = rows[i, sl] * a
                        return carry2
                    lax.fori_loop(0, CH // 16, scale_body, 0)

                    @pl.when(c == 0)
                    def _():
                        def efscale_body(g, carry2):
                            av = exb[pl.ds(g * 16, 16)]
                            for l in range(16):
                                i = g * 16 + l
                                sefb[i, :] = efb[i, :] * av[l]
                            return carry2
                        lax.fori_loop(0, CH // 16, efscale_body, 0)

                    @pl.when(t < SB // 2 - 1)
                    def _():
                        issue_in(kk + 2, b)

                    issue_out(kk, b)
                return carry
            lax.fori_loop(0, SB // 2, pair_body, 0)

            drain_out(SB - 2, 0)
            drain_out(SB - 1, 1)
            return carry0
        lax.fori_loop(0, NSUP, sup_body, 0)
        plsc.subcore_barrier()

        for t in range(8):
            sl = pl.ds(s * NSL + t * (NSL // 8), NSL // 8)
            pltpu.sync_copy(z1_sh.at[sl], cp_v)
            pltpu.sync_copy(cp_v, z1p_hbm.at[c, sl])
        for t in range(2):
            sl = pl.ds(s * NSL + t * (NSL // 2), NSL // 2)
            pltpu.sync_copy(q_sh.at[sl], qz_v)
            pltpu.sync_copy(qz_v, qp_hbm.at[c, sl])
        pltpu.sync_copy(den_sh.at[pl.ds(s * NSL, NSL)], dz_v)
        pltpu.sync_copy(dz_v, dp_hbm.at[c, 0, pl.ds(s * NSL, NSL)])

    return k(srcB, dstB, s1p, s2p, m1h, ef)


@jax.jit
def kernel(nfeats, efeats, edge_index, lin_w, attn_w, attn_b, Wapply_w, Wapply_b):
    nf = nfeats.reshape(N, DIN)
    ef = efeats.reshape(E, DE)
    srcB = edge_index[0].reshape(NS, RPT, CH)
    dstB = edge_index[1].reshape(NS, RPT, CH)

    w1 = lin_w[:, :DIN]
    w2 = lin_w[:, DIN:]
    a12 = attn_w.reshape(2, DIN).T
    ab = jnp.stack([attn_b[0], jnp.float32(0.0)]).reshape(1, 2)
    wa1 = Wapply_w[:, :DIN]
    wa2 = Wapply_w[:, DIN:]

    m1, s12 = _tc_prep(nf, w1, a12, ab)
    s1p = jnp.pad(s12[:, 0], (0, NPAD - N))
    s2p = jnp.pad(s12[:, 1], (0, NPAD - N))
    m1h = jnp.stack([m1[:, :DH], m1[:, DH:]])

    z1p, qp, dp = _sc_pass(srcB, dstB, s1p, s2p, m1h, ef)
    den_col = dp[1, 0, :N].reshape(N, 1)

    out = _tc_apply(nf, z1p, qp, den_col, w2, wa1, wa2,
                    Wapply_b.reshape(1, DOUT))
    return out.reshape(N, 1, DOUT)


# speedup vs baseline: 18.0927x; 1.0134x over previous
import functools

import jax
import jax.numpy as jnp
from jax import lax
from jax.experimental import pallas as pl
from jax.experimental.pallas import tpu as pltpu
from jax.experimental.pallas import tpu_sc as plsc

N = 10000
E = 320000
DIN = 128
DE = 16
DOUT = 128

ROW_BLK = 1000

NC = 2
NS = 16
NPAD = 10240
NSL = NPAD // NS
CH = 80
DH = DOUT // 2
EPT = E // NS
RPT = EPT // CH
SB = 50
NSUP = RPT // SB


def _z16():
    return jnp.zeros((16,), jnp.float32)


def _prep_body(nf_ref, w1_ref, a12_ref, ab_ref, m1h_ref, s1_ref, s2_ref):
    nf = nf_ref[...]
    m1 = lax.dot_general(nf, w1_ref[...], (((1,), (1,)), ((), ())),
                         preferred_element_type=jnp.float32)
    m1h_ref[0] = m1[:, :DOUT // 2]
    m1h_ref[1] = m1[:, DOUT // 2:]
    s12 = jnp.dot(nf, a12_ref[...],
                  preferred_element_type=jnp.float32) + ab_ref[...]
    s1_ref[...] = s12[:, 0:1]
    s2_ref[...] = s12[:, 1:2]


def _tc_prep(nf, w1, a12, ab):
    return pl.pallas_call(
        _prep_body,
        grid=(N // ROW_BLK,),
        in_specs=[
            pl.BlockSpec((ROW_BLK, DIN), lambda i: (i, 0)),
            pl.BlockSpec((DOUT, DIN), lambda i: (0, 0)),
            pl.BlockSpec((DIN, 2), lambda i: (0, 0)),
            pl.BlockSpec((1, 2), lambda i: (0, 0)),
        ],
        out_specs=[
            pl.BlockSpec((NC, ROW_BLK, DOUT // 2), lambda i: (0, i, 0)),
            pl.BlockSpec((ROW_BLK, 1), lambda i: (i, 0)),
            pl.BlockSpec((ROW_BLK, 1), lambda i: (i, 0)),
        ],
        out_shape=[
            jax.ShapeDtypeStruct((NC, N, DOUT // 2), jnp.float32),
            jax.ShapeDtypeStruct((NPAD, 1), jnp.float32),
            jax.ShapeDtypeStruct((NPAD, 1), jnp.float32),
        ],
    )(nf, w1, a12, ab)


def _apply_body(nf_ref, z1p_ref, qp_ref, den_ref, w2_ref, wa1_ref, wa2_ref,
                wb_ref, out_ref):
    zt = jnp.concatenate([z1p_ref[0], z1p_ref[1]], axis=-1)
    zt = zt + lax.dot_general(qp_ref[0], w2_ref[...], (((1,), (1,)), ((), ())),
                              preferred_element_type=jnp.float32)
    den = den_ref[...]
    z = zt / jnp.where(den > 0.0, den, 1.0)
    acc = lax.dot_general(nf_ref[...], wa1_ref[...], (((1,), (1,)), ((), ())),
                          preferred_element_type=jnp.float32)
    acc = acc + lax.dot_general(z, wa2_ref[...], (((1,), (1,)), ((), ())),
                                preferred_element_type=jnp.float32)
    out_ref[...] = jnp.maximum(acc + wb_ref[...], 0.0)


def _tc_apply(nf, z1p, qp, den_col, w2, wa1, wa2, wb):
    return pl.pallas_call(
        _apply_body,
        grid=(N // ROW_BLK,),
        in_specs=[
            pl.BlockSpec((ROW_BLK, DIN), lambda i: (i, 0)),
            pl.BlockSpec((NC, ROW_BLK, DH), lambda i: (0, i, 0)),
            pl.BlockSpec((NC, ROW_BLK, DE), lambda i: (0, i, 0)),
            pl.BlockSpec((ROW_BLK, 1), lambda i: (i, 0)),
            pl.BlockSpec((DOUT, DE), lambda i: (0, 0)),
            pl.BlockSpec((DOUT, DIN), lambda i: (0, 0)),
            pl.BlockSpec((DOUT, DOUT), lambda i: (0, 0)),
            pl.BlockSpec((1, DOUT), lambda i: (0, 0)),
        ],
        out_specs=pl.BlockSpec((ROW_BLK, DOUT), lambda i: (i, 0)),
        out_shape=jax.ShapeDtypeStruct((N, DOUT), jnp.float32),
    )(nf, z1p, qp, den_col, w2, wa1, wa2, wb)


def _sc_pass(srcB, dstB, s1p, s2p, m1h, ef):
    mesh = plsc.VectorSubcoreMesh(core_axis_name="c", subcore_axis_name="s")

    @functools.partial(
        pl.kernel,
        out_type=[
            jax.ShapeDtypeStruct((NC, NPAD, DH), jnp.float32),
            jax.ShapeDtypeStruct((NC, NPAD, DE), jnp.float32),
            jax.ShapeDtypeStruct((NC, 1, NPAD), jnp.float32),
        ],
        mesh=mesh,
        compiler_params=pltpu.CompilerParams(
            needs_layout_passes=False, use_tc_tiling_on_sc=False),
        scratch_types=[
            pltpu.VMEM((SB, CH), jnp.int32),
            pltpu.VMEM((SB, CH), jnp.int32),
            pltpu.VMEM((NPAD,), jnp.float32),
            pltpu.VMEM((NPAD,), jnp.float32),
            pltpu.VMEM((CH, DH), jnp.float32),
            pltpu.VMEM((CH, DH), jnp.float32),
            pltpu.VMEM((CH, DH), jnp.float32),
            pltpu.VMEM((CH, DH), jnp.float32),
            pltpu.VMEM((CH, DE), jnp.float32),
            pltpu.VMEM((CH, DE), jnp.float32),
            pltpu.VMEM((CH, DE), jnp.float32),
            pltpu.VMEM((CH, DE), jnp.float32),
            pltpu.VMEM((CH,), jnp.float32),
            pltpu.VMEM((CH,), jnp.float32),
            pltpu.VMEM((NSL // 8, DH), jnp.float32),
            pltpu.VMEM((NSL // 2, DE), jnp.float32),
            pltpu.VMEM((NSL,), jnp.float32),
            pltpu.VMEM_SHARED((NPAD, DH), jnp.float32),
            pltpu.VMEM_SHARED((NPAD, DE), jnp.float32),
            pltpu.VMEM_SHARED((NPAD,), jnp.float32),
            pltpu.SemaphoreType.DMA,
            pltpu.SemaphoreType.DMA,
            pltpu.SemaphoreType.DMA,
            pltpu.SemaphoreType.DMA,
        ],
    )
    def k(src_hbm, dst_hbm, s1_hbm, s2_hbm, m1h_hbm, ef_hbm,
          z1p_hbm, qp_hbm, dp_hbm,
          src_v, dst_v, s1_v, s2_v,
          rows0_v, rows1_v, srow0_v, srow1_v,
          ef0_v, ef1_v, sef0_v, sef1_v, ex0_v, ex1_v,
          cp_v, qz_v, dz_v, z1_sh, q_sh, den_sh,
          isem0, isem1, osem0, osem1):
        c = lax.axis_index("c")
        s = lax.axis_index("s")
        pltpu.sync_copy(s1_hbm, s1_v)
        pltpu.sync_copy(s2_hbm, s2_v)

        def zrow_body(r, carry):
            for j in range(DH // 16):
                cp_v[r, pl.ds(j * 16, 16)] = _z16()
            return carry
        lax.fori_loop(0, NSL // 8, zrow_body, 0)

        def zq_body(r, carry):
            qz_v[r, :] = _z16()
            return carry
        lax.fori_loop(0, NSL // 2, zq_body, 0)

        def zd_body(i, carry):
            dz_v[pl.ds(i * 16, 16)] = _z16()
            return carry
        lax.fori_loop(0, NSL // 16, zd_body, 0)

        for t in range(8):
            pltpu.sync_copy(
                cp_v, z1_sh.at[pl.ds(s * NSL + t * (NSL // 8), NSL // 8)])
        for t in range(2):
            pltpu.sync_copy(
                qz_v, q_sh.at[pl.ds(s * NSL + t * (NSL // 2), NSL // 2)])
        pltpu.sync_copy(dz_v, den_sh.at[pl.ds(s * NSL, NSL)])
        plsc.subcore_barrier()

        bufs = ((rows0_v, srow0_v, ef0_v, sef0_v, ex0_v, isem0, osem0),
                (rows1_v, srow1_v, ef1_v, sef1_v, ex1_v, isem1, osem1))

        def sup_body(m, carry0):
            pltpu.sync_copy(src_hbm.at[s, pl.ds(m * SB, SB)], src_v)
            pltpu.sync_copy(dst_hbm.at[s, pl.ds(m * SB, SB)], dst_v)

            def issue_in(kk, b):
                rows, _, efb, _, _, isem, _ = bufs[b]
                pltpu.async_copy(m1h_hbm.at[c].at[src_v.at[kk]], rows, isem)

                @pl.when(c == 0)
                def _():
                    pltpu.async_copy(
                        ef_hbm.at[pl.ds(s * EPT + (m * SB + kk) * CH, CH)],
                        efb, isem)

            def wait_in(kk, b):
                rows, _, efb, _, _, isem, _ = bufs[b]
                pltpu.make_async_copy(
                    m1h_hbm.at[c].at[src_v.at[kk]], rows, isem).wait()

                @pl.when(c == 0)
                def _():
                    pltpu.make_async_copy(
                        ef_hbm.at[pl.ds(s * EPT + (m * SB + kk) * CH, CH)],
                        efb, isem).wait()

            def issue_out(kk, b):
                _, srow, _, sefb, exb, _, osem = bufs[b]
                pltpu.async_copy(srow, z1_sh.at[dst_v.at[kk]], osem, add=True)

                @pl.when(c == 0)
                def _():
                    pltpu.async_copy(sefb, q_sh.at[dst_v.at[kk]], osem,
                                     add=True)

                @pl.when(c == 1)
                def _():
                    pltpu.async_copy(exb, den_sh.at[dst_v.at[kk]], osem,
                                     add=True)

            def drain_out(kk, b):
                _, srow, _, sefb, exb, _, osem = bufs[b]
                pltpu.make_async_copy(
                    srow, z1_sh.at[dst_v.at[kk]], osem).wait()

                @pl.when(c == 0)
                def _():
                    pltpu.make_async_copy(
                        sefb, q_sh.at[dst_v.at[kk]], osem).wait()

                @pl.when(c == 1)
                def _():
                    pltpu.make_async_copy(
                        exb, den_sh.at[dst_v.at[kk]], osem).wait()

            issue_in(0, 0)
            issue_in(1, 1)

            def pair_body(t, carry):
                for b in range(2):
                    kk = t * 2 + b
                    rows, srow, efb, sefb, exb, isem, osem = bufs[b]

                    @pl.when(t >= 1)
                    def _():
                        drain_out(kk - 2, b)

                    wait_in(kk, b)

                    for j in range(CH // 16):
                        i_s = src_v[kk, pl.ds(j * 16, 16)]
                        i_d = dst_v[kk, pl.ds(j * 16, 16)]
                        v = (plsc.load_gather(s1_v, [i_s])
                             + plsc.load_gather(s2_v, [i_d]))
                        v = jnp.where(v >= 0.0, v, v * 0.01)
                        exb[pl.ds(j * 16, 16)] = jnp.exp(v)

                    def scale_body(g, carry2):
                        av = exb[pl.ds(g * 16, 16)]
                        for l in range(16):
                            a = av[l]
                            i = g * 16 + l
                            for j in range(DH // 16):
                                sl = pl.ds(j * 16, 16)
                                srow[i, sl] = rows[i, sl] * a
                        return carry2
                    lax.fori_loop(0, CH // 16, scale_body, 0)

                    @pl.when(c == 0)
                    def _():
                        def efscale_body(g, carry2):
                            av = exb[pl.ds(g * 16, 16)]
                            for l in range(16):
                                i = g * 16 + l
                                sefb[i, :] = efb[i, :] * av[l]
                            return carry2
                        lax.fori_loop(0, CH // 16, efscale_body, 0)

                    @pl.when(t < SB // 2 - 1)
                    def _():
                        issue_in(kk + 2, b)

                    issue_out(kk, b)
                return carry
            lax.fori_loop(0, SB // 2, pair_body, 0)

            drain_out(SB - 2, 0)
            drain_out(SB - 1, 1)
            return carry0
        lax.fori_loop(0, NSUP, sup_body, 0)
        plsc.subcore_barrier()

        for t in range(8):
            sl = pl.ds(s * NSL + t * (NSL // 8), NSL // 8)
            pltpu.sync_copy(z1_sh.at[sl], cp_v)
            pltpu.sync_copy(cp_v, z1p_hbm.at[c, sl])
        for t in range(2):
            sl = pl.ds(s * NSL + t * (NSL // 2), NSL // 2)
            pltpu.sync_copy(q_sh.at[sl], qz_v)
            pltpu.sync_copy(qz_v, qp_hbm.at[c, sl])
        pltpu.sync_copy(den_sh.at[pl.ds(s * NSL, NSL)], dz_v)
        pltpu.sync_copy(dz_v, dp_hbm.at[c, 0, pl.ds(s * NSL, NSL)])

    return k(srcB, dstB, s1p, s2p, m1h, ef)


@jax.jit
def kernel(nfeats, efeats, edge_index, lin_w, attn_w, attn_b, Wapply_w, Wapply_b):
    nf = nfeats.reshape(N, DIN)
    ef = efeats.reshape(E, DE)
    srcB = edge_index[0].reshape(NS, RPT, CH)
    dstB = edge_index[1].reshape(NS, RPT, CH)

    w1 = lin_w[:, :DIN]
    w2 = lin_w[:, DIN:]
    a12 = attn_w.reshape(2, DIN).T
    ab = jnp.stack([attn_b[0], jnp.float32(0.0)]).reshape(1, 2)
    wa1 = Wapply_w[:, :DIN]
    wa2 = Wapply_w[:, DIN:]

    m1h, s1o, s2o = _tc_prep(nf, w1, a12, ab)
    s1p = s1o.reshape(NPAD)
    s2p = s2o.reshape(NPAD)

    z1p, qp, dp = _sc_pass(srcB, dstB, s1p, s2p, m1h, ef)
    den_col = dp[1, 0, :N].reshape(N, 1)

    out = _tc_apply(nf, z1p, qp, den_col, w2, wa1, wa2,
                    Wapply_b.reshape(1, DOUT))
    return out.reshape(N, 1, DOUT)


# speedup vs baseline: 18.1106x; 1.0010x over previous
import functools

import jax
import jax.numpy as jnp
from jax import lax
from jax.experimental import pallas as pl
from jax.experimental.pallas import tpu as pltpu
from jax.experimental.pallas import tpu_sc as plsc

N = 10000
E = 320000
DIN = 128
DE = 16
DOUT = 128

ROW_BLK = 1000

NC = 2
NS = 16
NPAD = 10240
NSL = NPAD // NS
CH = 80
DH = DOUT // 2
EPT = E // NS
RPT = EPT // CH
SB = 50
NSUP = RPT // SB


def _z16():
    return jnp.zeros((16,), jnp.float32)


def _prep_body(nf_ref, w1_ref, a12_ref, ab_ref, m1h_ref, s1_ref, s2_ref):
    nf = nf_ref[...]
    m1 = lax.dot_general(nf, w1_ref[...], (((1,), (1,)), ((), ())),
                         preferred_element_type=jnp.float32)
    m1h_ref[0] = m1[:, :DOUT // 2]
    m1h_ref[1] = m1[:, DOUT // 2:]
    s12 = jnp.dot(nf, a12_ref[...],
                  preferred_element_type=jnp.float32) + ab_ref[...]
    s1_ref[...] = s12[:, 0:1]
    s2_ref[...] = s12[:, 1:2]


def _tc_prep(nf, w1, a12, ab):
    return pl.pallas_call(
        _prep_body,
        grid=(N // ROW_BLK,),
        in_specs=[
            pl.BlockSpec((ROW_BLK, DIN), lambda i: (i, 0)),
            pl.BlockSpec((DOUT, DIN), lambda i: (0, 0)),
            pl.BlockSpec((DIN, 2), lambda i: (0, 0)),
            pl.BlockSpec((1, 2), lambda i: (0, 0)),
        ],
        out_specs=[
            pl.BlockSpec((NC, ROW_BLK, DOUT // 2), lambda i: (0, i, 0)),
            pl.BlockSpec((ROW_BLK, 1), lambda i: (i, 0)),
            pl.BlockSpec((ROW_BLK, 1), lambda i: (i, 0)),
        ],
        out_shape=[
            jax.ShapeDtypeStruct((NC, N, DOUT // 2), jnp.float32),
            jax.ShapeDtypeStruct((NPAD, 1), jnp.float32),
            jax.ShapeDtypeStruct((NPAD, 1), jnp.float32),
        ],
    )(nf, w1, a12, ab)


def _apply_body(nf_ref, z1p_ref, qp_ref, den_ref, w2_ref, wa1_ref, wa2_ref,
                wb_ref, out_ref):
    zt = jnp.concatenate([z1p_ref[0], z1p_ref[1]], axis=-1)
    zt = zt + lax.dot_general(qp_ref[0], w2_ref[...], (((1,), (1,)), ((), ())),
                              preferred_element_type=jnp.float32)
    den = den_ref[...]
    z = zt / jnp.where(den > 0.0, den, 1.0)
    acc = lax.dot_general(nf_ref[...], wa1_ref[...], (((1,), (1,)), ((), ())),
                          preferred_element_type=jnp.float32)
    acc = acc + lax.dot_general(z, wa2_ref[...], (((1,), (1,)), ((), ())),
                                preferred_element_type=jnp.float32)
    out_ref[...] = jnp.maximum(acc + wb_ref[...], 0.0)


def _tc_apply(nf, z1p, qp, den_col, w2, wa1, wa2, wb):
    return pl.pallas_call(
        _apply_body,
        grid=(N // ROW_BLK,),
        in_specs=[
            pl.BlockSpec((ROW_BLK, DIN), lambda i: (i, 0)),
            pl.BlockSpec((NC, ROW_BLK, DH), lambda i: (0, i, 0)),
            pl.BlockSpec((NC, ROW_BLK, DE), lambda i: (0, i, 0)),
            pl.BlockSpec((ROW_BLK, 1), lambda i: (i, 0)),
            pl.BlockSpec((DOUT, DE), lambda i: (0, 0)),
            pl.BlockSpec((DOUT, DIN), lambda i: (0, 0)),
            pl.BlockSpec((DOUT, DOUT), lambda i: (0, 0)),
            pl.BlockSpec((1, DOUT), lambda i: (0, 0)),
        ],
        out_specs=pl.BlockSpec((ROW_BLK, DOUT), lambda i: (i, 0)),
        out_shape=jax.ShapeDtypeStruct((N, DOUT), jnp.float32),
    )(nf, z1p, qp, den_col, w2, wa1, wa2, wb)


def _sc_pass(srcB, dstB, s1p, s2p, m1h, ef):
    mesh = plsc.VectorSubcoreMesh(core_axis_name="c", subcore_axis_name="s")

    @functools.partial(
        pl.kernel,
        out_type=[
            jax.ShapeDtypeStruct((NC, NPAD, DH), jnp.float32),
            jax.ShapeDtypeStruct((NC, NPAD, DE), jnp.float32),
            jax.ShapeDtypeStruct((NC, 1, NPAD), jnp.float32),
        ],
        mesh=mesh,
        compiler_params=pltpu.CompilerParams(
            needs_layout_passes=False, use_tc_tiling_on_sc=False),
        scratch_types=[
            pltpu.VMEM((SB, CH), jnp.int32),
            pltpu.VMEM((SB, CH), jnp.int32),
            pltpu.VMEM((NPAD,), jnp.float32),
            pltpu.VMEM((NPAD,), jnp.float32),
            pltpu.VMEM((CH, DH), jnp.float32),
            pltpu.VMEM((CH, DH), jnp.float32),
            pltpu.VMEM((CH, DH), jnp.float32),
            pltpu.VMEM((CH, DH), jnp.float32),
            pltpu.VMEM((CH, DE), jnp.float32),
            pltpu.VMEM((CH, DE), jnp.float32),
            pltpu.VMEM((CH, DE), jnp.float32),
            pltpu.VMEM((CH, DE), jnp.float32),
            pltpu.VMEM((CH,), jnp.float32),
            pltpu.VMEM((CH,), jnp.float32),
            pltpu.VMEM((NSL // 8, DH), jnp.float32),
            pltpu.VMEM((NSL // 2, DE), jnp.float32),
            pltpu.VMEM((NSL,), jnp.float32),
            pltpu.VMEM_SHARED((NPAD, DH), jnp.float32),
            pltpu.VMEM_SHARED((NPAD, DE), jnp.float32),
            pltpu.VMEM_SHARED((NPAD,), jnp.float32),
            pltpu.SemaphoreType.DMA,
            pltpu.SemaphoreType.DMA,
            pltpu.SemaphoreType.DMA,
            pltpu.SemaphoreType.DMA,
        ],
    )
    def k(src_hbm, dst_hbm, s1_hbm, s2_hbm, m1h_hbm, ef_hbm,
          z1p_hbm, qp_hbm, dp_hbm,
          src_v, dst_v, s1_v, s2_v,
          rows0_v, rows1_v, srow0_v, srow1_v,
          ef0_v, ef1_v, sef0_v, sef1_v, ex0_v, ex1_v,
          cp_v, qz_v, dz_v, z1_sh, q_sh, den_sh,
          isem0, isem1, osem0, osem1):
        c = lax.axis_index("c")
        s = lax.axis_index("s")
        pltpu.sync_copy(s1_hbm, s1_v)
        pltpu.sync_copy(s2_hbm, s2_v)

        def zrow_body(r, carry):
            for j in range(DH // 16):
                cp_v[r, pl.ds(j * 16, 16)] = _z16()
            return carry
        lax.fori_loop(0, NSL // 8, zrow_body, 0)

        def zq_body(r, carry):
            qz_v[r, :] = _z16()
            return carry
        lax.fori_loop(0, NSL // 2, zq_body, 0)

        def zd_body(i, carry):
            dz_v[pl.ds(i * 16, 16)] = _z16()
            return carry
        lax.fori_loop(0, NSL // 16, zd_body, 0)

        for t in range(8):
            pltpu.sync_copy(
                cp_v, z1_sh.at[pl.ds(s * NSL + t * (NSL // 8), NSL // 8)])
        for t in range(2):
            pltpu.sync_copy(
                qz_v, q_sh.at[pl.ds(s * NSL + t * (NSL // 2), NSL // 2)])
        pltpu.sync_copy(dz_v, den_sh.at[pl.ds(s * NSL, NSL)])
        plsc.subcore_barrier()

        bufs = ((rows0_v, srow0_v, ef0_v, sef0_v, ex0_v, isem0, osem0),
                (rows1_v, srow1_v, ef1_v, sef1_v, ex1_v, isem1, osem1))

        def sup_body(m, carry0):
            pltpu.sync_copy(src_hbm.at[s, pl.ds(m * SB, SB)], src_v)
            pltpu.sync_copy(dst_hbm.at[s, pl.ds(m * SB, SB)], dst_v)

            def issue_in(kk, b):
                rows, _, efb, _, _, isem, _ = bufs[b]
                pltpu.async_copy(m1h_hbm.at[c].at[src_v.at[kk]], rows, isem)

                @pl.when(c == 0)
                def _():
                    pltpu.async_copy(
                        ef_hbm.at[pl.ds(s * EPT + (m * SB + kk) * CH, CH), 0],
                        efb, isem)

            def wait_in(kk, b):
                rows, _, efb, _, _, isem, _ = bufs[b]
                pltpu.make_async_copy(
                    m1h_hbm.at[c].at[src_v.at[kk]], rows, isem).wait()

                @pl.when(c == 0)
                def _():
                    pltpu.make_async_copy(
                        ef_hbm.at[pl.ds(s * EPT + (m * SB + kk) * CH, CH), 0],
                        efb, isem).wait()

            def issue_out(kk, b):
                _, srow, _, sefb, exb, _, osem = bufs[b]
                pltpu.async_copy(srow, z1_sh.at[dst_v.at[kk]], osem, add=True)

                @pl.when(c == 0)
                def _():
                    pltpu.async_copy(sefb, q_sh.at[dst_v.at[kk]], osem,
                                     add=True)

                @pl.when(c == 1)
                def _():
                    pltpu.async_copy(exb, den_sh.at[dst_v.at[kk]], osem,
                                     add=True)

            def drain_out(kk, b):
                _, srow, _, sefb, exb, _, osem = bufs[b]
                pltpu.make_async_copy(
                    srow, z1_sh.at[dst_v.at[kk]], osem).wait()

                @pl.when(c == 0)
                def _():
                    pltpu.make_async_copy(
                        sefb, q_sh.at[dst_v.at[kk]], osem).wait()

                @pl.when(c == 1)
                def _():
                    pltpu.make_async_copy(
                        exb, den_sh.at[dst_v.at[kk]], osem).wait()

            issue_in(0, 0)
            issue_in(1, 1)

            def pair_body(t, carry):
                for b in range(2):
                    kk = t * 2 + b
                    rows, srow, efb, sefb, exb, isem, osem = bufs[b]

                    @pl.when(t >= 1)
                    def _():
                        drain_out(kk - 2, b)

                    wait_in(kk, b)

                    for j in range(CH // 16):
                        i_s = src_v[kk, pl.ds(j * 16, 16)]
                        i_d = dst_v[kk, pl.ds(j * 16, 16)]
                        v = (plsc.load_gather(s1_v, [i_s])
                             + plsc.load_gather(s2_v, [i_d]))
                        v = jnp.where(v >= 0.0, v, v * 0.01)
                        exb[pl.ds(j * 16, 16)] = jnp.exp(v)

                    def scale_body(g, carry2):
                        av = exb[pl.ds(g * 16, 16)]
                        for l in range(16):
                            a = av[l]
                            i = g * 16 + l
                            for j in range(DH // 16):
                                sl = pl.ds(j * 16, 16)
                                srow[i, sl] = rows[i, sl] * a
                        return carry2
                    lax.fori_loop(0, CH // 16, scale_body, 0)

                    @pl.when(c == 0)
                    def _():
                        def efscale_body(g, carry2):
                            av = exb[pl.ds(g * 16, 16)]
                            for l in range(16):
                                i = g * 16 + l
                                sefb[i, :] = efb[i, :] * av[l]
                            return carry2
                        lax.fori_loop(0, CH // 16, efscale_body, 0)

                    @pl.when(t < SB // 2 - 1)
                    def _():
                        issue_in(kk + 2, b)

                    issue_out(kk, b)
                return carry
            lax.fori_loop(0, SB // 2, pair_body, 0)

            drain_out(SB - 2, 0)
            drain_out(SB - 1, 1)
            return carry0
        lax.fori_loop(0, NSUP, sup_body, 0)
        plsc.subcore_barrier()

        for t in range(8):
            sl = pl.ds(s * NSL + t * (NSL // 8), NSL // 8)
            pltpu.sync_copy(z1_sh.at[sl], cp_v)
            pltpu.sync_copy(cp_v, z1p_hbm.at[c, sl])
        for t in range(2):
            sl = pl.ds(s * NSL + t * (NSL // 2), NSL // 2)
            pltpu.sync_copy(q_sh.at[sl], qz_v)
            pltpu.sync_copy(qz_v, qp_hbm.at[c, sl])
        pltpu.sync_copy(den_sh.at[pl.ds(s * NSL, NSL)], dz_v)
        pltpu.sync_copy(dz_v, dp_hbm.at[c, 0, pl.ds(s * NSL, NSL)])

    return k(srcB, dstB, s1p, s2p, m1h, ef)


@jax.jit
def kernel(nfeats, efeats, edge_index, lin_w, attn_w, attn_b, Wapply_w, Wapply_b):
    nf = nfeats.reshape(N, DIN)
    ef = efeats
    srcB = edge_index[0].reshape(NS, RPT, CH)
    dstB = edge_index[1].reshape(NS, RPT, CH)

    w1 = lin_w[:, :DIN]
    w2 = lin_w[:, DIN:]
    a12 = attn_w.reshape(2, DIN).T
    ab = jnp.stack([attn_b[0], jnp.float32(0.0)]).reshape(1, 2)
    wa1 = Wapply_w[:, :DIN]
    wa2 = Wapply_w[:, DIN:]

    m1h, s1o, s2o = _tc_prep(nf, w1, a12, ab)
    s1p = s1o.reshape(NPAD)
    s2p = s2o.reshape(NPAD)

    z1p, qp, dp = _sc_pass(srcB, dstB, s1p, s2p, m1h, ef)
    den_col = dp[1, 0, :N].reshape(N, 1)

    out = _tc_apply(nf, z1p, qp, den_col, w2, wa1, wa2,
                    Wapply_b.reshape(1, DOUT))
    return out.reshape(N, 1, DOUT)


# speedup vs baseline: 18.1125x; 1.0001x over previous
import functools

import jax
import jax.numpy as jnp
from jax import lax
from jax.experimental import pallas as pl
from jax.experimental.pallas import tpu as pltpu
from jax.experimental.pallas import tpu_sc as plsc

N = 10000
E = 320000
DIN = 128
DE = 16
DOUT = 128

ROW_BLK = 1000

NC = 2
NS = 16
NPAD = 10240
NSL = NPAD // NS
CH = 80
DH = DOUT // 2
EPT = E // NS
RPT = EPT // CH
SB = 50
NSUP = RPT // SB
EFR = CH * DE // 128


def _z16():
    return jnp.zeros((16,), jnp.float32)


def _prep_body(nf_ref, w1_ref, a12_ref, ab_ref, m1h_ref, s1_ref, s2_ref):
    nf = nf_ref[...]
    m1 = lax.dot_general(nf, w1_ref[...], (((1,), (1,)), ((), ())),
                         preferred_element_type=jnp.float32)
    m1h_ref[0] = m1[:, :DOUT // 2]
    m1h_ref[1] = m1[:, DOUT // 2:]
    s12 = jnp.dot(nf, a12_ref[...],
                  preferred_element_type=jnp.float32) + ab_ref[...]
    s1_ref[...] = s12[:, 0:1]
    s2_ref[...] = s12[:, 1:2]


def _tc_prep(nf, w1, a12, ab):
    return pl.pallas_call(
        _prep_body,
        grid=(N // ROW_BLK,),
        in_specs=[
            pl.BlockSpec((ROW_BLK, DIN), lambda i: (i, 0)),
            pl.BlockSpec((DOUT, DIN), lambda i: (0, 0)),
            pl.BlockSpec((DIN, 2), lambda i: (0, 0)),
            pl.BlockSpec((1, 2), lambda i: (0, 0)),
        ],
        out_specs=[
            pl.BlockSpec((NC, ROW_BLK, DOUT // 2), lambda i: (0, i, 0)),
            pl.BlockSpec((ROW_BLK, 1), lambda i: (i, 0)),
            pl.BlockSpec((ROW_BLK, 1), lambda i: (i, 0)),
        ],
        out_shape=[
            jax.ShapeDtypeStruct((NC, N, DOUT // 2), jnp.float32),
            jax.ShapeDtypeStruct((NPAD, 1), jnp.float32),
            jax.ShapeDtypeStruct((NPAD, 1), jnp.float32),
        ],
    )(nf, w1, a12, ab)


def _apply_body(nf_ref, z1p_ref, qp_ref, den_ref, w2_ref, wa1_ref, wa2_ref,
                wb_ref, out_ref):
    zt = jnp.concatenate([z1p_ref[0], z1p_ref[1]], axis=-1)
    zt = zt + lax.dot_general(qp_ref[0], w2_ref[...], (((1,), (1,)), ((), ())),
                              preferred_element_type=jnp.float32)
    den = den_ref[...]
    z = zt / jnp.where(den > 0.0, den, 1.0)
    acc = lax.dot_general(nf_ref[...], wa1_ref[...], (((1,), (1,)), ((), ())),
                          preferred_element_type=jnp.float32)
    acc = acc + lax.dot_general(z, wa2_ref[...], (((1,), (1,)), ((), ())),
                                preferred_element_type=jnp.float32)
    out_ref[...] = jnp.maximum(acc + wb_ref[...], 0.0)


def _tc_apply(nf, z1p, qp, den_col, w2, wa1, wa2, wb):
    return pl.pallas_call(
        _apply_body,
        grid=(N // ROW_BLK,),
        in_specs=[
            pl.BlockSpec((ROW_BLK, DIN), lambda i: (i, 0)),
            pl.BlockSpec((NC, ROW_BLK, DH), lambda i: (0, i, 0)),
            pl.BlockSpec((NC, ROW_BLK, DE), lambda i: (0, i, 0)),
            pl.BlockSpec((ROW_BLK, 1), lambda i: (i, 0)),
            pl.BlockSpec((DOUT, DE), lambda i: (0, 0)),
            pl.BlockSpec((DOUT, DIN), lambda i: (0, 0)),
            pl.BlockSpec((DOUT, DOUT), lambda i: (0, 0)),
            pl.BlockSpec((1, DOUT), lambda i: (0, 0)),
        ],
        out_specs=pl.BlockSpec((ROW_BLK, DOUT), lambda i: (i, 0)),
        out_shape=jax.ShapeDtypeStruct((N, DOUT), jnp.float32),
    )(nf, z1p, qp, den_col, w2, wa1, wa2, wb)


def _sc_pass(srcB, dstB, s1p, s2p, m1h, ef):
    mesh = plsc.VectorSubcoreMesh(core_axis_name="c", subcore_axis_name="s")

    @functools.partial(
        pl.kernel,
        out_type=[
            jax.ShapeDtypeStruct((NC, NPAD, DH), jnp.float32),
            jax.ShapeDtypeStruct((NC, NPAD, DE), jnp.float32),
            jax.ShapeDtypeStruct((NC, 1, NPAD), jnp.float32),
        ],
        mesh=mesh,
        compiler_params=pltpu.CompilerParams(
            needs_layout_passes=False, use_tc_tiling_on_sc=False),
        scratch_types=[
            pltpu.VMEM((SB, CH), jnp.int32),
            pltpu.VMEM((SB, CH), jnp.int32),
            pltpu.VMEM((NPAD,), jnp.float32),
            pltpu.VMEM((NPAD,), jnp.float32),
            pltpu.VMEM((CH, DH), jnp.float32),
            pltpu.VMEM((CH, DH), jnp.float32),
            pltpu.VMEM((CH, DH), jnp.float32),
            pltpu.VMEM((CH, DH), jnp.float32),
            pltpu.VMEM((EFR, 128), jnp.float32),
            pltpu.VMEM((EFR, 128), jnp.float32),
            pltpu.VMEM((CH, DE), jnp.float32),
            pltpu.VMEM((CH, DE), jnp.float32),
            pltpu.VMEM((CH,), jnp.float32),
            pltpu.VMEM((CH,), jnp.float32),
            pltpu.VMEM((NSL // 8, DH), jnp.float32),
            pltpu.VMEM((NSL // 2, DE), jnp.float32),
            pltpu.VMEM((NSL,), jnp.float32),
            pltpu.VMEM_SHARED((NPAD, DH), jnp.float32),
            pltpu.VMEM_SHARED((NPAD, DE), jnp.float32),
            pltpu.VMEM_SHARED((NPAD,), jnp.float32),
            pltpu.SemaphoreType.DMA,
            pltpu.SemaphoreType.DMA,
            pltpu.SemaphoreType.DMA,
            pltpu.SemaphoreType.DMA,
        ],
    )
    def k(src_hbm, dst_hbm, s1_hbm, s2_hbm, m1h_hbm, ef_hbm,
          z1p_hbm, qp_hbm, dp_hbm,
          src_v, dst_v, s1_v, s2_v,
          rows0_v, rows1_v, srow0_v, srow1_v,
          ef0_v, ef1_v, sef0_v, sef1_v, ex0_v, ex1_v,
          cp_v, qz_v, dz_v, z1_sh, q_sh, den_sh,
          isem0, isem1, osem0, osem1):
        c = lax.axis_index("c")
        s = lax.axis_index("s")
        pltpu.sync_copy(s1_hbm, s1_v)
        pltpu.sync_copy(s2_hbm, s2_v)

        def zrow_body(r, carry):
            for j in range(DH // 16):
                cp_v[r, pl.ds(j * 16, 16)] = _z16()
            return carry
        lax.fori_loop(0, NSL // 8, zrow_body, 0)

        def zq_body(r, carry):
            qz_v[r, :] = _z16()
            return carry
        lax.fori_loop(0, NSL // 2, zq_body, 0)

        def zd_body(i, carry):
            dz_v[pl.ds(i * 16, 16)] = _z16()
            return carry
        lax.fori_loop(0, NSL // 16, zd_body, 0)

        for t in range(8):
            pltpu.sync_copy(
                cp_v, z1_sh.at[pl.ds(s * NSL + t * (NSL // 8), NSL // 8)])
        for t in range(2):
            pltpu.sync_copy(
                qz_v, q_sh.at[pl.ds(s * NSL + t * (NSL // 2), NSL // 2)])
        pltpu.sync_copy(dz_v, den_sh.at[pl.ds(s * NSL, NSL)])
        plsc.subcore_barrier()

        bufs = ((rows0_v, srow0_v, ef0_v, sef0_v, ex0_v, isem0, osem0),
                (rows1_v, srow1_v, ef1_v, sef1_v, ex1_v, isem1, osem1))

        def sup_body(m, carry0):
            pltpu.sync_copy(src_hbm.at[s, pl.ds(m * SB, SB)], src_v)
            pltpu.sync_copy(dst_hbm.at[s, pl.ds(m * SB, SB)], dst_v)

            def issue_in(kk, b):
                rows, _, efb, _, _, isem, _ = bufs[b]
                pltpu.async_copy(m1h_hbm.at[c].at[src_v.at[kk]], rows, isem)

                @pl.when(c == 0)
                def _():
                    pltpu.async_copy(
                        ef_hbm.at[pl.ds((s * EPT + (m * SB + kk) * CH)
                                        * DE // 128, EFR)],
                        efb, isem)

            def wait_in(kk, b):
                rows, _, efb, _, _, isem, _ = bufs[b]
                pltpu.make_async_copy(
                    m1h_hbm.at[c].at[src_v.at[kk]], rows, isem).wait()

                @pl.when(c == 0)
                def _():
                    pltpu.make_async_copy(
                        ef_hbm.at[pl.ds((s * EPT + (m * SB + kk) * CH)
                                        * DE // 128, EFR)],
                        efb, isem).wait()

            def issue_out(kk, b):
                _, srow, _, sefb, exb, _, osem = bufs[b]
                pltpu.async_copy(srow, z1_sh.at[dst_v.at[kk]], osem, add=True)

                @pl.when(c == 0)
                def _():
                    pltpu.async_copy(sefb, q_sh.at[dst_v.at[kk]], osem,
                                     add=True)

                @pl.when(c == 1)
                def _():
                    pltpu.async_copy(exb, den_sh.at[dst_v.at[kk]], osem,
                                     add=True)

            def drain_out(kk, b):
                _, srow, _, sefb, exb, _, osem = bufs[b]
                pltpu.make_async_copy(
                    srow, z1_sh.at[dst_v.at[kk]], osem).wait()

                @pl.when(c == 0)
                def _():
                    pltpu.make_async_copy(
                        sefb, q_sh.at[dst_v.at[kk]], osem).wait()

                @pl.when(c == 1)
                def _():
                    pltpu.make_async_copy(
                        exb, den_sh.at[dst_v.at[kk]], osem).wait()

            issue_in(0, 0)
            issue_in(1, 1)

            def pair_body(t, carry):
                for b in range(2):
                    kk = t * 2 + b
                    rows, srow, efb, sefb, exb, isem, osem = bufs[b]

                    @pl.when(t >= 1)
                    def _():
                        drain_out(kk - 2, b)

                    wait_in(kk, b)

                    for j in range(CH // 16):
                        i_s = src_v[kk, pl.ds(j * 16, 16)]
                        i_d = dst_v[kk, pl.ds(j * 16, 16)]
                        v = (plsc.load_gather(s1_v, [i_s])
                             + plsc.load_gather(s2_v, [i_d]))
                        v = jnp.where(v >= 0.0, v, v * 0.01)
                        exb[pl.ds(j * 16, 16)] = jnp.exp(v)

                    def scale_body(g, carry2):
                        av = exb[pl.ds(g * 16, 16)]
                        for l in range(16):
                            a = av[l]
                            i = g * 16 + l
                            for j in range(DH // 16):
                                sl = pl.ds(j * 16, 16)
                                srow[i, sl] = rows[i, sl] * a
                        return carry2
                    lax.fori_loop(0, CH // 16, scale_body, 0)

                    @pl.when(c == 0)
                    def _():
                        def efscale_body(g, carry2):
                            av = exb[pl.ds(g * 16, 16)]
                            for l in range(16):
                                i = g * 16 + l
                                sefb[i, :] = (
                                    efb[2 * g + l // 8,
                                        pl.ds((l % 8) * 16, 16)] * av[l])
                            return carry2
                        lax.fori_loop(0, CH // 16, efscale_body, 0)

                    @pl.when(t < SB // 2 - 1)
                    def _():
                        issue_in(kk + 2, b)

                    issue_out(kk, b)
                return carry
            lax.fori_loop(0, SB // 2, pair_body, 0)

            drain_out(SB - 2, 0)
            drain_out(SB - 1, 1)
            return carry0
        lax.fori_loop(0, NSUP, sup_body, 0)
        plsc.subcore_barrier()

        for t in range(8):
            sl = pl.ds(s * NSL + t * (NSL // 8), NSL // 8)
            pltpu.sync_copy(z1_sh.at[sl], cp_v)
            pltpu.sync_copy(cp_v, z1p_hbm.at[c, sl])
        for t in range(2):
            sl = pl.ds(s * NSL + t * (NSL // 2), NSL // 2)
            pltpu.sync_copy(q_sh.at[sl], qz_v)
            pltpu.sync_copy(qz_v, qp_hbm.at[c, sl])
        pltpu.sync_copy(den_sh.at[pl.ds(s * NSL, NSL)], dz_v)
        pltpu.sync_copy(dz_v, dp_hbm.at[c, 0, pl.ds(s * NSL, NSL)])

    return k(srcB, dstB, s1p, s2p, m1h, ef)


@jax.jit
def kernel(nfeats, efeats, edge_index, lin_w, attn_w, attn_b, Wapply_w, Wapply_b):
    nf = nfeats.reshape(N, DIN)
    ef = efeats.reshape(E * DE // 128, 128)
    srcB = edge_index[0].reshape(NS, RPT, CH)
    dstB = edge_index[1].reshape(NS, RPT, CH)

    w1 = lin_w[:, :DIN]
    w2 = lin_w[:, DIN:]
    a12 = attn_w.reshape(2, DIN).T
    ab = jnp.stack([attn_b[0], jnp.float32(0.0)]).reshape(1, 2)
    wa1 = Wapply_w[:, :DIN]
    wa2 = Wapply_w[:, DIN:]

    m1h, s1o, s2o = _tc_prep(nf, w1, a12, ab)
    s1p = s1o.reshape(NPAD)
    s2p = s2o.reshape(NPAD)

    z1p, qp, dp = _sc_pass(srcB, dstB, s1p, s2p, m1h, ef)
    den_col = dp[1, 0, :N].reshape(N, 1)

    out = _tc_apply(nf, z1p, qp, den_col, w2, wa1, wa2,
                    Wapply_b.reshape(1, DOUT))
    return out.reshape(N, 1, DOUT)
